# baseline (speedup 1.0000x reference)
import functools

import jax
import jax.numpy as jnp
from jax import lax
from jax.experimental import pallas as pl
from jax.experimental.pallas import tpu as pltpu
from jax.experimental.pallas import tpu_sc as plsc

_DIM = 64
_PAD = 128
_CHUNK = 128
_K = 2
_GRP = _K * _CHUNK
_LANES = 16
_NC = 2
_NS = 16
_NW = _NC * _NS


def _embed_body(table_hbm, idx_hbm, out_hbm, idx_v, gbuf, tbuf, gsem, wsem):
    wid = lax.axis_index("s") * _NC + lax.axis_index("c")
    rows_total = idx_hbm.shape[0]
    per_w = rows_total // _NW
    base = wid * per_w
    seq = out_hbm.shape[2]
    n_groups = per_w // _K
    iota = lax.iota(jnp.int32, _LANES)

    pltpu.sync_copy(idx_hbm.at[pl.ds(base, per_w)], idx_v)

    def fire_gathers(g, s):
        for j in range(_K):
            pltpu.async_copy(
                table_hbm.at[idx_v.at[g * _K + j]],
                gbuf.at[s, pl.ds(j * _CHUNK, _CHUNK)],
                gsem,
            )

    def wait_gathers(s):
        pltpu.make_async_copy(
            table_hbm.at[pl.ds(0, _GRP)], gbuf.at[s], gsem
        ).wait()

    def fire_writeback(g, slab):
        t0 = (base + g * _K) * _CHUNK + slab * _CHUNK
        b = t0 // seq
        off = t0 % seq
        pltpu.async_copy(
            tbuf.at[slab], out_hbm.at[b, :, pl.ds(off, _CHUNK)], wsem
        )

    def wait_writeback(slab):
        pltpu.make_async_copy(
            tbuf.at[slab], out_hbm.at[0, :, pl.ds(0, _CHUNK)], wsem
        ).wait()

    def transpose_slab(s, slab, rowvs):
        gslot = gbuf.at[s]
        nk = _CHUNK // _LANES
        for c in range(_DIM):
            colv = jnp.full((_LANES,), c, jnp.int32)
            vals = [plsc.load_gather(gslot, [rowvs[slab][k], colv]) for k in range(nk)]
            for k in range(nk):
                tbuf[slab, c, pl.ds(k * _LANES, _LANES)] = vals[k]

    rowvs = [
        [iota + (slab * _CHUNK + k * _LANES) for k in range(_CHUNK // _LANES)]
        for slab in range(_K)
    ]

    fire_gathers(0, 0)

    def pair_body(i, carry):
        for s in (0, 1):
            g = 2 * i + s

            @pl.when(g + 1 < n_groups)
            def _fire_next():
                fire_gathers(g + 1, 1 - s)

            wait_gathers(s)
            for slab in range(_K):

                @pl.when(g >= 1)
                def _free_slab():
                    wait_writeback(slab)

                transpose_slab(s, slab, rowvs)
                fire_writeback(g, slab)
        return carry

    lax.fori_loop(0, n_groups // 2, pair_body, 0)
    wait_writeback(0)
    wait_writeback(1)


@jax.jit
def _gather_rows(table_pad, idx2d):
    mesh = plsc.VectorSubcoreMesh(core_axis_name="c", subcore_axis_name="s")
    n_rows = idx2d.shape[0]
    batch = (n_rows * _CHUNK) // 1024
    fn = functools.partial(
        pl.kernel,
        mesh=mesh,
        out_type=jax.ShapeDtypeStruct((batch, _DIM, 1024), jnp.float32),
        scratch_types=[
            pltpu.VMEM((n_rows // _NW, _CHUNK), jnp.int32),
            pltpu.VMEM((2, _GRP, _PAD), jnp.float32),
            pltpu.VMEM((_K, _DIM, _CHUNK), jnp.float32),
            pltpu.SemaphoreType.DMA,
            pltpu.SemaphoreType.DMA,
        ],
        compiler_params=pltpu.CompilerParams(
            use_tc_tiling_on_sc=True, needs_layout_passes=False
        ),
    )(_embed_body)
    return fn(table_pad, idx2d)


def kernel(input_ids, attention_mask, table):
    b, s = input_ids.shape
    n_rows = (b * s) // _CHUNK
    idx2d = input_ids.reshape(n_rows, _CHUNK)
    table_pad = jnp.pad(table, ((0, 0), (0, _PAD - _DIM)))
    out_t = _gather_rows(table_pad, idx2d)
    token_embeddings = jnp.transpose(out_t, (0, 2, 1))
    return (input_ids, token_embeddings, attention_mask)


# speedup vs baseline: 1.3273x; 1.3273x over previous
import functools

import jax
import jax.numpy as jnp
from jax import lax
from jax.experimental import pallas as pl
from jax.experimental.pallas import tpu as pltpu
from jax.experimental.pallas import tpu_sc as plsc

_DIM = 64
_CHUNK = 128
_K = 4
_NC = 2
_NS = 16
_NW = _NC * _NS


def _embed_body(table_hbm, idx_hbm, out_hbm, idx_v, rows_v, gsem, wsem):
    wid = lax.axis_index("s") * _NC + lax.axis_index("c")
    rows_total = idx_hbm.shape[0]
    per_w = rows_total // _NW
    base = wid * per_w
    n_groups = per_w // _K

    pltpu.sync_copy(idx_hbm.at[pl.ds(base, per_w)], idx_v)

    def fire_gathers(g, s):
        for j in range(_K):
            pltpu.async_copy(
                table_hbm.at[idx_v.at[g * _K + j]], rows_v.at[s].at[j], gsem
            )

    def wait_gathers(s):
        pltpu.make_async_copy(out_hbm.at[pl.ds(0, _K)], rows_v.at[s], gsem).wait()

    def fire_writeback(g, s):
        pltpu.async_copy(rows_v.at[s], out_hbm.at[pl.ds(base + g * _K, _K)], wsem)

    def wait_writeback(s):
        pltpu.make_async_copy(rows_v.at[s], out_hbm.at[pl.ds(0, _K)], wsem).wait()

    fire_gathers(0, 0)

    def pair_body(i, carry):
        for s in (0, 1):
            g = 2 * i + s
            nxt_exists = g + 1 < n_groups

            @pl.when(nxt_exists)
            def _fire_next():
                if s == 1:
                    wait_writeback(1 - s)
                else:

                    @pl.when(g >= 1)
                    def _():
                        wait_writeback(1 - s)

                fire_gathers(g + 1, 1 - s)

            wait_gathers(s)
            fire_writeback(g, s)
        return carry

    lax.fori_loop(0, n_groups // 2, pair_body, 0)
    wait_writeback(0)
    wait_writeback(1)


@functools.partial(jax.jit, static_argnames=("n_rows",))
def _gather_rows(table, idx2d, n_rows):
    mesh = plsc.VectorSubcoreMesh(core_axis_name="c", subcore_axis_name="s")
    fn = functools.partial(
        pl.kernel,
        mesh=mesh,
        out_type=jax.ShapeDtypeStruct((n_rows, _CHUNK, _DIM), jnp.float32),
        scratch_types=[
            pltpu.VMEM((n_rows // _NW, _CHUNK), jnp.int32),
            pltpu.VMEM((2, _K, _CHUNK, _DIM), jnp.float32),
            pltpu.SemaphoreType.DMA,
            pltpu.SemaphoreType.DMA,
        ],
        compiler_params=pltpu.CompilerParams(use_tc_tiling_on_sc=False),
    )(_embed_body)
    return fn(table, idx2d)


def kernel(input_ids, attention_mask, table):
    b, s = input_ids.shape
    total = b * s
    n_rows = total // _CHUNK
    idx2d = input_ids.reshape(n_rows, _CHUNK)
    out3d = _gather_rows(table, idx2d, n_rows)
    token_embeddings = out3d.reshape(b, s, _DIM)
    return (input_ids, token_embeddings, attention_mask)


# speedup vs baseline: 1.6477x; 1.2413x over previous
import functools

import jax
import jax.numpy as jnp
from jax import lax
from jax.experimental import pallas as pl
from jax.experimental.pallas import tpu as pltpu
from jax.experimental.pallas import tpu_sc as plsc

_DIM = 64
_PAD = 128
_CHUNK = 128
_K = 2
_NC = 2
_NS = 16
_NW = _NC * _NS


def _embed_body(table_hbm, idx_hbm, out_hbm, idx_v, gbuf, gsem, wsem):
    wid = lax.axis_index("s") * _NC + lax.axis_index("c")
    rows_total = idx_hbm.shape[0]
    per_w = rows_total // _NW
    base = wid * per_w
    seq = out_hbm.shape[1]
    grp = _K * _CHUNK
    n_groups = per_w // _K

    pltpu.sync_copy(idx_hbm.at[pl.ds(base, per_w)], idx_v)

    def fire_gathers(g, s):
        for j in range(_K):
            pltpu.async_copy(
                table_hbm.at[idx_v.at[g * _K + j]],
                gbuf.at[s, pl.ds(j * _CHUNK, _CHUNK)],
                gsem,
            )

    def wait_gathers(s):
        pltpu.make_async_copy(table_hbm.at[pl.ds(0, grp)], gbuf.at[s], gsem).wait()

    def fire_writeback(g, s):
        t0 = (base + g * _K) * _CHUNK
        b = t0 // seq
        s0 = t0 % seq
        pltpu.async_copy(gbuf.at[s], out_hbm.at[b, pl.ds(s0, grp)], wsem)

    def wait_writeback(s):
        pltpu.make_async_copy(gbuf.at[s], out_hbm.at[0, pl.ds(0, grp)], wsem).wait()

    fire_gathers(0, 0)

    def pair_body(i, carry):
        for s in (0, 1):
            g = 2 * i + s

            @pl.when(g + 1 < n_groups)
            def _fire_next():
                if s == 1:
                    wait_writeback(1 - s)
                else:

                    @pl.when(g >= 1)
                    def _():
                        wait_writeback(1 - s)

                fire_gathers(g + 1, 1 - s)

            wait_gathers(s)
            fire_writeback(g, s)
        return carry

    lax.fori_loop(0, n_groups // 2, pair_body, 0)
    wait_writeback(0)
    wait_writeback(1)


@functools.partial(jax.jit, static_argnames=("batch", "seq"))
def _gather_rows(table_pad, idx2d, batch, seq):
    mesh = plsc.VectorSubcoreMesh(core_axis_name="c", subcore_axis_name="s")
    n_rows = idx2d.shape[0]
    fn = functools.partial(
        pl.kernel,
        mesh=mesh,
        out_type=jax.ShapeDtypeStruct((batch, seq, _PAD), jnp.float32),
        scratch_types=[
            pltpu.VMEM((n_rows // _NW, _CHUNK), jnp.int32),
            pltpu.VMEM((2, _K * _CHUNK, _PAD), jnp.float32),
            pltpu.SemaphoreType.DMA,
            pltpu.SemaphoreType.DMA,
        ],
        compiler_params=pltpu.CompilerParams(use_tc_tiling_on_sc=False),
    )(_embed_body)
    return fn(table_pad, idx2d)


def kernel(input_ids, attention_mask, table):
    b, s = input_ids.shape
    n_rows = (b * s) // _CHUNK
    idx2d = input_ids.reshape(n_rows, _CHUNK)
    table_pad = jnp.pad(table, ((0, 0), (0, _PAD - _DIM)))
    out128 = _gather_rows(table_pad, idx2d, b, s)
    token_embeddings = out128[:, :, :_DIM]
    return (input_ids, token_embeddings, attention_mask)
